# Initial kernel scaffold; baseline (speedup 1.0000x reference)
import jax
import jax.numpy as jnp
from jax.experimental import pallas as pl


def kernel(feats, coords):
    raise NotImplementedError("write your pallas kernel here")


# speedup vs baseline: 3.9852x; 3.9852x over previous
import functools

import jax
import jax.numpy as jnp
from jax import lax
from jax.experimental import pallas as pl
from jax.experimental.pallas import tpu as pltpu
from jax.experimental.pallas import tpu_sc as plsc

G = 2048
N_PTS = 1_000_000
C = 2048
D = 128
ND = C // D
NC, NS = 2, 16
NW = NC * NS
NCHUNK = (N_PTS + C - 1) // C
K_ITERS = (NCHUNK + NW - 1) // NW
GRID_W = (G + 2) * G

_mesh = plsc.VectorSubcoreMesh(
    core_axis_name="c", subcore_axis_name="s", num_cores=NC, num_subcores=NS
)


def _compute_idx(xb, yb, idxb, row_off):
  def vec(j, carry):
    xv = xb[pl.ds(j * 16, 16)]
    yv = yb[pl.ds(j * 16, 16)]
    idxb[j // 8, pl.ds((j % 8) * 16, 16)] = (xv + row_off) * G + yv
    return carry
  lax.fori_loop(0, C // 16, vec, 0, unroll=4)


@functools.partial(
    pl.kernel,
    out_type=(),
    mesh=_mesh,
    scratch_types=[
        pltpu.VMEM((C,), jnp.int32),
        pltpu.VMEM((C,), jnp.int32),
        pltpu.VMEM((C,), jnp.float32),
        pltpu.VMEM((ND, D), jnp.int32),
        pltpu.SemaphoreType.DMA,
    ],
)
def _scatter(x_hbm, y_hbm, f_hbm, grid_ref, xb, yb, fb, idxb, sem):
  wid = lax.axis_index("s") * NC + lax.axis_index("c")

  def chunk(k, carry):
    cid = wid + NW * k

    @pl.when(cid < NCHUNK)
    def _():
      base = jnp.minimum(cid * C, N_PTS - C)
      cx = pltpu.async_copy(x_hbm.at[pl.ds(base, C)], xb, sem)
      cy = pltpu.async_copy(y_hbm.at[pl.ds(base, C)], yb, sem)
      cf = pltpu.async_copy(f_hbm.at[pl.ds(base, C)], fb, sem)
      cx.wait(); cy.wait(); cf.wait()
      _compute_idx(xb, yb, idxb, 1)
      descs = [
          pltpu.async_copy(
              fb.at[pl.ds(d * D, D)], grid_ref.at[idxb.at[d]], sem)
          for d in range(ND)
      ]
      for dsc in descs:
        dsc.wait()

    return carry

  lax.fori_loop(0, K_ITERS, chunk, 0)


@functools.partial(
    pl.kernel,
    out_type=jax.ShapeDtypeStruct((N_PTS,), jnp.float32),
    mesh=_mesh,
    scratch_types=[
        pltpu.VMEM((C,), jnp.int32),
        pltpu.VMEM((C,), jnp.int32),
        pltpu.VMEM((ND, D), jnp.int32),
        pltpu.VMEM((ND, D), jnp.int32),
        pltpu.VMEM((C,), jnp.float32),
        pltpu.VMEM((C,), jnp.float32),
        pltpu.VMEM((C,), jnp.float32),
        pltpu.SemaphoreType.DMA,
    ],
)
def _gather(x_hbm, y_hbm, grid_ref, out_hbm,
            xb, yb, idxPb, idxMb, gPb, gMb, ob, sem):
  wid = lax.axis_index("s") * NC + lax.axis_index("c")

  def chunk(k, carry):
    cid = wid + NW * k

    @pl.when(cid < NCHUNK)
    def _():
      base = jnp.minimum(cid * C, N_PTS - C)
      cx = pltpu.async_copy(x_hbm.at[pl.ds(base, C)], xb, sem)
      cy = pltpu.async_copy(y_hbm.at[pl.ds(base, C)], yb, sem)
      cx.wait(); cy.wait()
      _compute_idx(xb, yb, idxPb, 2)
      _compute_idx(xb, yb, idxMb, 0)
      descs = [
          pltpu.async_copy(
              grid_ref.at[idxPb.at[d]], gPb.at[pl.ds(d * D, D)], sem)
          for d in range(ND)
      ] + [
          pltpu.async_copy(
              grid_ref.at[idxMb.at[d]], gMb.at[pl.ds(d * D, D)], sem)
          for d in range(ND)
      ]
      for dsc in descs:
        dsc.wait()

      def vec(j, c2):
        gp = gPb[pl.ds(j * 16, 16)]
        gm = gMb[pl.ds(j * 16, 16)]
        ob[pl.ds(j * 16, 16)] = 0.5 * (gp - gm)
        return c2
      lax.fori_loop(0, C // 16, vec, 0, unroll=4)
      pltpu.sync_copy(ob, out_hbm.at[pl.ds(base, C)])

    return carry

  lax.fori_loop(0, K_ITERS, chunk, 0)


def kernel(feats, coords):
  x = coords[:, 0].astype(jnp.int32)
  y = coords[:, 1].astype(jnp.int32)
  f = feats[:, 0]
  grid_ref = jax.new_ref(jnp.zeros((GRID_W,), jnp.float32))
  _scatter(x, y, f, grid_ref)
  out = _gather(x, y, grid_ref)
  return out[:, None]
